# Initial kernel scaffold; baseline (speedup 1.0000x reference)
import jax
import jax.numpy as jnp
from jax.experimental import pallas as pl


def kernel(x, relative_positions):
    raise NotImplementedError("write your pallas kernel here")


# speedup vs baseline: 7.2336x; 7.2336x over previous
import functools

import jax
import jax.numpy as jnp
from jax import lax
from jax.experimental import pallas as pl
from jax.experimental.pallas import tpu as pltpu
from jax.experimental.pallas import tpu_sc as plsc

NUM_CORES = 2
NUM_SUBCORES = 16
LANES = 16
NW = NUM_CORES * NUM_SUBCORES


def _build_sc_call(seq: int, table_rows: int, hid: int, chunk: int):
    max_len = (table_rows + 1) // 2
    rows_per_w = seq // NW
    n_chunks = seq // chunk
    groups = chunk // LANES

    mesh = plsc.VectorSubcoreMesh(
        core_axis_name="c", subcore_axis_name="s",
        num_cores=NUM_CORES, num_subcores=NUM_SUBCORES)

    @functools.partial(
        pl.kernel,
        out_type=jax.ShapeDtypeStruct((seq, seq, hid), jnp.float32),
        mesh=mesh,
        scratch_types=[
            pltpu.VMEM((chunk,), jnp.int32),
            pltpu.VMEM((chunk, hid), jnp.float32),
            pltpu.SemaphoreType.DMA,
        ],
    )
    def sc_gather(table_hbm, out_hbm, idx_ref, buf_ref, sem):
        wid = lax.axis_index("s") * NUM_CORES + lax.axis_index("c")
        lane = lax.iota(jnp.int32, LANES)

        def task(t, _):
            i = wid * rows_per_w + t // n_chunks
            j0 = (t % n_chunks) * chunk
            base = i - j0 + (max_len - 1)

            def fill(g, _):
                idx_ref[pl.ds(g * LANES, LANES)] = (
                    jnp.full((LANES,), base, jnp.int32) - g * LANES - lane)
                return 0
            lax.fori_loop(0, groups, fill, 0)

            pltpu.async_copy(table_hbm.at[idx_ref], buf_ref, sem).wait()
            pltpu.sync_copy(buf_ref, out_hbm.at[i, pl.ds(j0, chunk), :])
            return 0

        lax.fori_loop(0, rows_per_w * n_chunks, task, 0)

    return sc_gather


def kernel(x, relative_positions):
    seq = x.shape[1]
    table_rows, hid = relative_positions.shape
    call = _build_sc_call(seq, table_rows, hid, chunk=256)
    return call(relative_positions)


# speedup vs baseline: 7.4259x; 1.0266x over previous
import functools

import jax
import jax.numpy as jnp
from jax import lax
from jax.experimental import pallas as pl
from jax.experimental.pallas import tpu as pltpu
from jax.experimental.pallas import tpu_sc as plsc

NUM_CORES = 2
NUM_SUBCORES = 16
LANES = 16
NW = NUM_CORES * NUM_SUBCORES
NBUF = 2


def _build_sc_call(seq: int, table_rows: int, hid: int, chunk: int):
    max_len = (table_rows + 1) // 2
    rows_per_w = seq // NW
    n_chunks = seq // chunk
    groups = chunk // LANES
    n_tasks = rows_per_w * n_chunks

    mesh = plsc.VectorSubcoreMesh(
        core_axis_name="c", subcore_axis_name="s",
        num_cores=NUM_CORES, num_subcores=NUM_SUBCORES)

    @functools.partial(
        pl.kernel,
        out_type=jax.ShapeDtypeStruct((seq, seq, hid), jnp.float32),
        mesh=mesh,
        scratch_types=[
            *[pltpu.VMEM((chunk,), jnp.int32) for _ in range(NBUF)],
            *[pltpu.VMEM((chunk, hid), jnp.float32) for _ in range(NBUF)],
            *[pltpu.SemaphoreType.DMA for _ in range(2 * NBUF)],
        ],
    )
    def sc_gather(table_hbm, out_hbm, *scr):
        idx = scr[:NBUF]
        buf = scr[NBUF:2 * NBUF]
        gsem = scr[2 * NBUF:3 * NBUF]
        wsem = scr[3 * NBUF:]
        wid = lax.axis_index("s") * NUM_CORES + lax.axis_index("c")
        lane = lax.iota(jnp.int32, LANES)
        row0 = wid * rows_per_w

        def task_coords(t):
            i = row0 + t // n_chunks
            j0 = (t % n_chunks) * chunk
            return i, j0

        def start_gather(t, b):
            i, j0 = task_coords(t)
            base = i - j0 + (max_len - 1)
            for g in range(groups):
                idx[b][pl.ds(g * LANES, LANES)] = (base - g * LANES) - lane
            pltpu.make_async_copy(table_hbm.at[idx[b]], buf[b], gsem[b]).start()

        for b in range(NBUF):
            start_gather(b, b)

        def body(k, _):
            for b in range(NBUF):
                t = NBUF * k + b
                i, j0 = task_coords(t)
                dst = out_hbm.at[i, pl.ds(j0, chunk), :]
                pltpu.make_async_copy(table_hbm.at[idx[b]], buf[b], gsem[b]).wait()
                wcopy = pltpu.make_async_copy(buf[b], dst, wsem[b])
                wcopy.start()
                wcopy.wait()

                @pl.when(t + NBUF < n_tasks)
                def _():
                    start_gather(t + NBUF, b)
            return 0

        lax.fori_loop(0, n_tasks // NBUF, body, 0)

    return sc_gather


def kernel(x, relative_positions):
    seq = x.shape[1]
    table_rows, hid = relative_positions.shape
    call = _build_sc_call(seq, table_rows, hid, chunk=256)
    return call(relative_positions)


# speedup vs baseline: 19.7001x; 2.6529x over previous
import functools

import jax
import jax.numpy as jnp
from jax import lax
from jax.experimental import pallas as pl
from jax.experimental.pallas import tpu as pltpu
from jax.experimental.pallas import tpu_sc as plsc

NUM_CORES = 2
NUM_SUBCORES = 16
LANES = 16
NW = NUM_CORES * NUM_SUBCORES


def _build_sc_call(seq: int, table_rows: int, hid: int, chunk: int):
    max_len = (table_rows + 1) // 2
    rows_per_w = seq // NW
    n_chunks = seq // chunk
    win = rows_per_w + chunk - 1
    win_pad = ((win + LANES - 1) // LANES) * LANES
    groups = win_pad // LANES

    mesh = plsc.VectorSubcoreMesh(
        core_axis_name="c", subcore_axis_name="s",
        num_cores=NUM_CORES, num_subcores=NUM_SUBCORES)

    @functools.partial(
        pl.kernel,
        out_type=jax.ShapeDtypeStruct((seq, seq, hid), jnp.float32),
        mesh=mesh,
        scratch_types=[
            pltpu.VMEM((win_pad,), jnp.int32),
            pltpu.VMEM((win_pad, hid), jnp.float32),
            pltpu.SemaphoreType.DMA,
            pltpu.SemaphoreType.DMA,
        ],
    )
    def sc_gather(table_hbm, out_hbm, idx, wbuf, gsem, wsem):
        wid = lax.axis_index("s") * NUM_CORES + lax.axis_index("c")
        lane = lax.iota(jnp.int32, LANES)
        i0 = wid * rows_per_w

        def block(jb, _):
            j0 = jb * chunk
            hi = i0 - j0 + (max_len - 1) + (rows_per_w - 1)
            for g in range(groups):
                idx[pl.ds(g * LANES, LANES)] = jnp.maximum(
                    (hi - g * LANES) - lane, 0)
            gcopy = pltpu.make_async_copy(table_hbm.at[idx], wbuf, gsem)
            gcopy.start()
            gcopy.wait()

            for di in range(rows_per_w):
                pltpu.make_async_copy(
                    wbuf.at[pl.ds(rows_per_w - 1 - di, chunk), :],
                    out_hbm.at[i0 + di, pl.ds(j0, chunk), :],
                    wsem).start()
            for di in range(rows_per_w):
                pltpu.make_async_copy(
                    wbuf.at[pl.ds(rows_per_w - 1 - di, chunk), :],
                    out_hbm.at[i0 + di, pl.ds(j0, chunk), :],
                    wsem).wait()
            return 0

        lax.fori_loop(0, n_chunks, block, 0)

    return sc_gather


def kernel(x, relative_positions):
    seq = x.shape[1]
    table_rows, hid = relative_positions.shape
    call = _build_sc_call(seq, table_rows, hid, chunk=512)
    return call(relative_positions)
